# baseline (speedup 1.0000x reference)
import functools

import jax
import jax.numpy as jnp
from jax import lax
from jax.experimental import pallas as pl
from jax.experimental.pallas import tpu as pltpu
from jax.experimental.pallas import tpu_sc as plsc

T = 2048
D = 768
E = 16
F = 1024
BLK = 256
NBLK = (2 * T) // BLK + E
CAP = NBLK * BLK
NC = 2
NS = 16
NW = NC * NS
TPW = T // NW


def _cumsum_rows(a, n):
    sh = 1
    while sh < n:
        z = jnp.zeros((sh, a.shape[1]), a.dtype)
        a = a + jnp.concatenate([z, a[:-sh, :]], axis=0)
        sh *= 2
    return a


def _route_body(x_ref, gw_ref, bias_ref, pos0_ref, pos1_ref, sb0_ref, sb1_ref,
                eb_ref, nact_ref):
    x = x_ref[...]
    gw = gw_ref[...]
    logits = lax.dot_general(x, gw, (((1,), (1,)), ((), ())),
                             preferred_element_type=jnp.float32,
                             precision=lax.Precision.HIGHEST)
    scores = jax.nn.sigmoid(logits + bias_ref[...])

    eidx = lax.broadcasted_iota(jnp.int32, (T, E), 1)
    m0 = jnp.max(scores, axis=1, keepdims=True)
    i0 = jnp.min(jnp.where(scores == m0, eidx, E), axis=1, keepdims=True)
    masked = jnp.where(eidx == i0, -1.0, scores)
    m1 = jnp.max(masked, axis=1, keepdims=True)
    i1 = jnp.min(jnp.where(masked == m1, eidx, E), axis=1, keepdims=True)
    denom = m0 + m1 + 1e-6
    s0 = m0 / denom
    s1 = m1 / denom
    sb0_ref[...] = jnp.broadcast_to(s0, (T, E))
    sb1_ref[...] = jnp.broadcast_to(s1, (T, E))

    oh0 = (eidx == i0).astype(jnp.float32)
    oh1 = (eidx == i1).astype(jnp.float32)
    c01 = _cumsum_rows(jnp.concatenate([oh0, oh1], axis=1), T)
    c0 = c01[:, :E]
    c1 = c01[:, E:]

    counts = jnp.sum(oh0 + oh1, axis=0, keepdims=True)
    nb = jnp.floor((counts + (BLK - 1)) * (1.0 / BLK))
    r16 = lax.broadcasted_iota(jnp.int32, (E, E), 0)
    cjj = lax.broadcasted_iota(jnp.int32, (E, E), 1)
    lt_strict = (r16 < cjj).astype(jnp.float32)
    offs = lax.dot_general(nb, lt_strict, (((1,), (0,)), ((), ())),
                           preferred_element_type=jnp.float32,
                           precision=lax.Precision.HIGHEST)
    offs_end = offs + nb
    offset_pad = offs * float(BLK)

    rank0 = jnp.sum(oh0 * (c0 - 1.0 + c1), axis=1, keepdims=True)
    rank1 = jnp.sum(oh1 * (c0 + c1 - 1.0), axis=1, keepdims=True)
    base0 = jnp.sum(oh0 * offset_pad, axis=1, keepdims=True)
    base1 = jnp.sum(oh1 * offset_pad, axis=1, keepdims=True)
    pos0_ref[...] = (base0 + rank0).astype(jnp.int32)
    pos1_ref[...] = (base1 + rank1).astype(jnp.int32)

    gidx = lax.broadcasted_iota(jnp.int32, (NBLK, 1), 0).astype(jnp.float32)
    ebf = jnp.sum((jnp.broadcast_to(offs_end, (NBLK, E)) <= gidx)
                  .astype(jnp.float32), axis=1, keepdims=True)
    e_last = jnp.max(jnp.where(nb > 0.0,
                               lax.broadcasted_iota(jnp.int32, (1, E), 1)
                               .astype(jnp.float32),
                               -1.0), axis=1, keepdims=True)
    e_last = jnp.maximum(e_last, 0.0)
    eb_ref[...] = jnp.minimum(ebf, jnp.broadcast_to(e_last, (NBLK, 1))).astype(jnp.int32)
    nact_ref[...] = offs_end[:, E - 1:E].astype(jnp.int32)


def _route(xf, gate_w, bias2d, interpret=False):
    return pl.pallas_call(
        _route_body,
        out_shape=(
            jax.ShapeDtypeStruct((T, 1), jnp.int32),
            jax.ShapeDtypeStruct((T, 1), jnp.int32),
            jax.ShapeDtypeStruct((T, E), jnp.float32),
            jax.ShapeDtypeStruct((T, E), jnp.float32),
            jax.ShapeDtypeStruct((NBLK, 1), jnp.int32),
            jax.ShapeDtypeStruct((1, 1), jnp.int32),
        ),
        interpret=interpret,
    )(xf, gate_w, bias2d)


def _ffn_body(eb_s, nact_s, xs_ref, w1_ref, w2_ref, y_ref):
    g = pl.program_id(0)

    @pl.when(g < nact_s[0])
    def _():
        xb = xs_ref[...].astype(jnp.bfloat16)
        h = lax.dot_general(xb, w1_ref[0].astype(jnp.bfloat16),
                            (((1,), (1,)), ((), ())),
                            preferred_element_type=jnp.float32)
        h = h * jax.nn.sigmoid(h)
        y_ref[...] = lax.dot_general(h.astype(jnp.bfloat16),
                                     w2_ref[0].astype(jnp.bfloat16),
                                     (((1,), (1,)), ((), ())),
                                     preferred_element_type=jnp.float32)


def _ffn(eb, nact, xs, w1, w2, interpret=False):
    grid_spec = pltpu.PrefetchScalarGridSpec(
        num_scalar_prefetch=2,
        grid=(NBLK,),
        in_specs=[
            pl.BlockSpec((BLK, D), lambda g, eb_s, nact_s: (g, 0)),
            pl.BlockSpec((1, F, D), lambda g, eb_s, nact_s: (eb_s[g], 0, 0)),
            pl.BlockSpec((1, D, F), lambda g, eb_s, nact_s: (eb_s[g], 0, 0)),
        ],
        out_specs=pl.BlockSpec((BLK, D), lambda g, eb_s, nact_s: (g, 0)),
    )
    return pl.pallas_call(
        _ffn_body,
        grid_spec=grid_spec,
        out_shape=jax.ShapeDtypeStruct((CAP, D), jnp.float32),
        interpret=interpret,
    )(eb, nact, xs, w1, w2)


@functools.cache
def _sc_kernels():
    mesh = plsc.VectorSubcoreMesh(core_axis_name="c", subcore_axis_name="s",
                                  num_cores=NC, num_subcores=NS)

    @functools.partial(
        pl.kernel,
        out_type=jax.ShapeDtypeStruct((CAP, D), jnp.float32),
        mesh=mesh,
        scratch_types=[
            pltpu.VMEM((TPW, D), jnp.float32),
            pltpu.VMEM((TPW,), jnp.int32),
            pltpu.VMEM((TPW,), jnp.int32),
            pltpu.SemaphoreType.DMA,
        ],
    )
    def _dispatch(x_hbm, pos0_hbm, pos1_hbm, out_hbm, rows_v, p0_v, p1_v, sem):
        wid = lax.axis_index("s") * NC + lax.axis_index("c")
        base = wid * TPW
        pltpu.sync_copy(x_hbm.at[pl.ds(base, TPW)], rows_v)
        pltpu.sync_copy(pos0_hbm.at[pl.ds(base, TPW)], p0_v)
        pltpu.sync_copy(pos1_hbm.at[pl.ds(base, TPW)], p1_v)
        pltpu.async_copy(rows_v, out_hbm.at[p0_v], sem).wait()
        pltpu.async_copy(rows_v, out_hbm.at[p1_v], sem).wait()

    @functools.partial(
        pl.kernel,
        out_type=jax.ShapeDtypeStruct((T, D), jnp.float32),
        mesh=mesh,
        scratch_types=[
            pltpu.VMEM((TPW, D), jnp.float32),
            pltpu.VMEM((TPW, D), jnp.float32),
            pltpu.VMEM((TPW,), jnp.int32),
            pltpu.VMEM((TPW,), jnp.int32),
            pltpu.VMEM((TPW, E), jnp.float32),
            pltpu.VMEM((TPW, E), jnp.float32),
            pltpu.SemaphoreType.DMA,
        ],
    )
    def _combine(ys_hbm, pos0_hbm, pos1_hbm, sb0_hbm, sb1_hbm, out_hbm,
                 y0_v, y1_v, p0_v, p1_v, s0_v, s1_v, sem):
        wid = lax.axis_index("s") * NC + lax.axis_index("c")
        base = wid * TPW
        pltpu.sync_copy(pos0_hbm.at[pl.ds(base, TPW)], p0_v)
        pltpu.sync_copy(pos1_hbm.at[pl.ds(base, TPW)], p1_v)
        pltpu.sync_copy(sb0_hbm.at[pl.ds(base, TPW)], s0_v)
        pltpu.sync_copy(sb1_hbm.at[pl.ds(base, TPW)], s1_v)
        pltpu.async_copy(ys_hbm.at[p0_v], y0_v, sem).wait()
        pltpu.async_copy(ys_hbm.at[p1_v], y1_v, sem).wait()

        def body(i, carry):
            s0r = s0_v[i, :]
            s1r = s1_v[i, :]
            for j in range(D // 16):
                sl = pl.ds(j * 16, 16)
                y0_v[i, sl] = y0_v[i, sl] * s0r + y1_v[i, sl] * s1r
            return carry

        lax.fori_loop(0, TPW, body, 0)
        pltpu.sync_copy(y0_v, out_hbm.at[pl.ds(base, TPW)])

    return _dispatch, _combine


def kernel(x, gate_w, w1, w2, balance_bias):
    b, s, d = x.shape
    xf = x.reshape(-1, d)
    bias2d = balance_bias.reshape(1, E)
    pos0, pos1, sb0, sb1, eb, nact = _route(xf, gate_w, bias2d)
    pos0 = pos0.reshape(-1)
    pos1 = pos1.reshape(-1)
    dispatch_fn, combine_fn = _sc_kernels()
    xs = dispatch_fn(xf, pos0, pos1)
    ys = _ffn(eb.reshape(-1), nact.reshape(-1), xs, w1, w2)
    out = combine_fn(ys, pos0, pos1, sb0, sb1)
    return out.reshape(b, s, d)


# speedup vs baseline: 1.1010x; 1.1010x over previous
import functools

import jax
import jax.numpy as jnp
from jax import lax
from jax.experimental import pallas as pl
from jax.experimental.pallas import tpu as pltpu
from jax.experimental.pallas import tpu_sc as plsc

T = 2048
D = 768
E = 16
F = 1024
BLK = 256
NBLK = (2 * T) // BLK + E
CAP = NBLK * BLK
NC = 2
NS = 16
NW = NC * NS
TPW = T // NW


def _cumsum_rows(a, n):
    sh = 1
    while sh < n:
        z = jnp.zeros((sh, a.shape[1]), a.dtype)
        a = a + jnp.concatenate([z, a[:-sh, :]], axis=0)
        sh *= 2
    return a


def _route_body(x_ref, gw_ref, bias_ref, pos0_ref, pos1_ref, sb0_ref, sb1_ref,
                eb_ref, chg_ref, slot_ref, nxt_ref, issue_ref, nact_ref):
    x = x_ref[...]
    gw = gw_ref[...]
    logits = lax.dot_general(x, gw, (((1,), (1,)), ((), ())),
                             preferred_element_type=jnp.float32)
    scores = jax.nn.sigmoid(logits + bias_ref[...])

    eidx = lax.broadcasted_iota(jnp.int32, (T, E), 1)
    m0 = jnp.max(scores, axis=1, keepdims=True)
    i0 = jnp.min(jnp.where(scores == m0, eidx, E), axis=1, keepdims=True)
    masked = jnp.where(eidx == i0, -1.0, scores)
    m1 = jnp.max(masked, axis=1, keepdims=True)
    i1 = jnp.min(jnp.where(masked == m1, eidx, E), axis=1, keepdims=True)
    denom = m0 + m1 + 1e-6
    s0 = m0 / denom
    s1 = m1 / denom
    sb0_ref[...] = jnp.broadcast_to(s0, (T, E))
    sb1_ref[...] = jnp.broadcast_to(s1, (T, E))

    oh0 = (eidx == i0).astype(jnp.float32)
    oh1 = (eidx == i1).astype(jnp.float32)
    c01 = _cumsum_rows(jnp.concatenate([oh0, oh1], axis=1), T)
    c0 = c01[:, :E]
    c1 = c01[:, E:]

    counts = jnp.sum(oh0 + oh1, axis=0, keepdims=True)
    nb = jnp.floor((counts + (BLK - 1)) * (1.0 / BLK))
    r16 = lax.broadcasted_iota(jnp.int32, (E, E), 0)
    cjj = lax.broadcasted_iota(jnp.int32, (E, E), 1)
    lt_strict = (r16 < cjj).astype(jnp.float32)
    offs = lax.dot_general(nb, lt_strict, (((1,), (0,)), ((), ())),
                           preferred_element_type=jnp.float32)
    offs_end = offs + nb
    offset_pad = offs * float(BLK)

    rank0 = jnp.sum(oh0 * (c0 - 1.0 + c1), axis=1, keepdims=True)
    rank1 = jnp.sum(oh1 * (c0 + c1 - 1.0), axis=1, keepdims=True)
    base0 = jnp.sum(oh0 * offset_pad, axis=1, keepdims=True)
    base1 = jnp.sum(oh1 * offset_pad, axis=1, keepdims=True)
    pos0_ref[...] = (base0 + rank0).astype(jnp.int32)
    pos1_ref[...] = (base1 + rank1).astype(jnp.int32)

    gidx = lax.broadcasted_iota(jnp.int32, (NBLK, 1), 0).astype(jnp.float32)
    ebf = jnp.sum((jnp.broadcast_to(offs_end, (NBLK, E)) <= gidx)
                  .astype(jnp.float32), axis=1, keepdims=True)
    e_last = jnp.max(jnp.where(nb > 0.0,
                               lax.broadcasted_iota(jnp.int32, (1, E), 1)
                               .astype(jnp.float32),
                               -1.0), axis=1, keepdims=True)
    e_last = jnp.maximum(e_last, 0.0)
    ebf = jnp.minimum(ebf, jnp.broadcast_to(e_last, (NBLK, 1)))
    eb_ref[...] = ebf.astype(jnp.int32)
    nact_ref[...] = offs_end[:, E - 1:E].astype(jnp.int32)

    ebprev = jnp.concatenate([jnp.full((1, 1), -1.0, jnp.float32),
                              ebf[:-1, :]], axis=0)
    chg = (ebf != ebprev).astype(jnp.float32)
    runid = _cumsum_rows(chg, NBLK) - 1.0
    slot = runid - 2.0 * jnp.floor(runid * 0.5)
    r_iota = lax.broadcasted_iota(jnp.int32, (1, NBLK), 1).astype(jnp.float32)
    m_run = (runid == r_iota).astype(jnp.float32) * chg
    eor = jnp.sum(m_run * ebf, axis=0, keepdims=True)
    nruns = jnp.max(runid, axis=0, keepdims=True) + 1.0
    nxt = jnp.sum(((runid + 1.0) == r_iota).astype(jnp.float32) * eor,
                  axis=1, keepdims=True)
    issue = chg * ((runid + 1.0) < nruns).astype(jnp.float32)
    chg_ref[...] = chg.astype(jnp.int32)
    slot_ref[...] = slot.astype(jnp.int32)
    nxt_ref[...] = nxt.astype(jnp.int32)
    issue_ref[...] = issue.astype(jnp.int32)


def _route(xf, gate_w, bias2d, interpret=False):
    return pl.pallas_call(
        _route_body,
        out_shape=(
            jax.ShapeDtypeStruct((T, 1), jnp.int32),
            jax.ShapeDtypeStruct((T, 1), jnp.int32),
            jax.ShapeDtypeStruct((T, E), jnp.float32),
            jax.ShapeDtypeStruct((T, E), jnp.float32),
            jax.ShapeDtypeStruct((NBLK, 1), jnp.int32),
            jax.ShapeDtypeStruct((NBLK, 1), jnp.int32),
            jax.ShapeDtypeStruct((NBLK, 1), jnp.int32),
            jax.ShapeDtypeStruct((NBLK, 1), jnp.int32),
            jax.ShapeDtypeStruct((NBLK, 1), jnp.int32),
            jax.ShapeDtypeStruct((1, 1), jnp.int32),
        ),
        interpret=interpret,
    )(xf, gate_w, bias2d)


def _ffn_body(eb_s, chg_s, slot_s, nxt_s, issue_s, nact_s,
              xs_ref, w1_hbm, w2_hbm, y_ref, w1b, w2b, sem1, sem2):
    g = pl.program_id(0)
    s = slot_s[g]
    cur = eb_s[g]

    @pl.when(g == 0)
    def _():
        pltpu.make_async_copy(w1_hbm.at[cur], w1b.at[s], sem1.at[s]).start()
        pltpu.make_async_copy(w2_hbm.at[cur], w2b.at[s], sem2.at[s]).start()

    @pl.when(chg_s[g] == 1)
    def _():
        pltpu.make_async_copy(w1_hbm.at[cur], w1b.at[s], sem1.at[s]).wait()
        pltpu.make_async_copy(w2_hbm.at[cur], w2b.at[s], sem2.at[s]).wait()

    @pl.when(issue_s[g] == 1)
    def _():
        nx = nxt_s[g]
        pltpu.make_async_copy(w1_hbm.at[nx], w1b.at[1 - s], sem1.at[1 - s]).start()
        pltpu.make_async_copy(w2_hbm.at[nx], w2b.at[1 - s], sem2.at[1 - s]).start()

    @pl.when(g < nact_s[0])
    def _():
        xb = xs_ref[...]
        h = lax.dot_general(xb, w1b[s], (((1,), (1,)), ((), ())),
                            preferred_element_type=jnp.float32)
        h = h * jax.nn.sigmoid(h)
        y_ref[...] = lax.dot_general(h, w2b[s], (((1,), (1,)), ((), ())),
                                     preferred_element_type=jnp.float32)


def _ffn(eb, chg, slot, nxt, issue, nact, xs, w1, w2, interpret=False):
    grid_spec = pltpu.PrefetchScalarGridSpec(
        num_scalar_prefetch=6,
        grid=(NBLK,),
        in_specs=[
            pl.BlockSpec((BLK, D), lambda g, *_: (g, 0)),
            pl.BlockSpec(memory_space=pltpu.MemorySpace.HBM),
            pl.BlockSpec(memory_space=pltpu.MemorySpace.HBM),
        ],
        out_specs=pl.BlockSpec((BLK, D), lambda g, *_: (g, 0)),
        scratch_shapes=[
            pltpu.VMEM((2, F, D), jnp.float32),
            pltpu.VMEM((2, D, F), jnp.float32),
            pltpu.SemaphoreType.DMA((2,)),
            pltpu.SemaphoreType.DMA((2,)),
        ],
    )
    return pl.pallas_call(
        _ffn_body,
        grid_spec=grid_spec,
        out_shape=jax.ShapeDtypeStruct((CAP, D), jnp.float32),
        interpret=interpret,
    )(eb, chg, slot, nxt, issue, nact, xs, w1, w2)


@functools.cache
def _sc_kernels():
    mesh = plsc.VectorSubcoreMesh(core_axis_name="c", subcore_axis_name="s",
                                  num_cores=NC, num_subcores=NS)

    @functools.partial(
        pl.kernel,
        out_type=jax.ShapeDtypeStruct((CAP, D), jnp.float32),
        mesh=mesh,
        scratch_types=[
            pltpu.VMEM((TPW, D), jnp.float32),
            pltpu.VMEM((TPW,), jnp.int32),
            pltpu.VMEM((TPW,), jnp.int32),
            pltpu.SemaphoreType.DMA,
        ],
    )
    def _dispatch(x_hbm, pos0_hbm, pos1_hbm, out_hbm, rows_v, p0_v, p1_v, sem):
        wid = lax.axis_index("s") * NC + lax.axis_index("c")
        base = wid * TPW
        pltpu.sync_copy(x_hbm.at[pl.ds(base, TPW)], rows_v)
        pltpu.sync_copy(pos0_hbm.at[pl.ds(base, TPW)], p0_v)
        pltpu.sync_copy(pos1_hbm.at[pl.ds(base, TPW)], p1_v)
        pltpu.async_copy(rows_v, out_hbm.at[p0_v], sem).wait()
        pltpu.async_copy(rows_v, out_hbm.at[p1_v], sem).wait()

    @functools.partial(
        pl.kernel,
        out_type=jax.ShapeDtypeStruct((T, D), jnp.float32),
        mesh=mesh,
        scratch_types=[
            pltpu.VMEM((TPW, D), jnp.float32),
            pltpu.VMEM((TPW, D), jnp.float32),
            pltpu.VMEM((TPW,), jnp.int32),
            pltpu.VMEM((TPW,), jnp.int32),
            pltpu.VMEM((TPW, E), jnp.float32),
            pltpu.VMEM((TPW, E), jnp.float32),
            pltpu.SemaphoreType.DMA,
        ],
    )
    def _combine(ys_hbm, pos0_hbm, pos1_hbm, sb0_hbm, sb1_hbm, out_hbm,
                 y0_v, y1_v, p0_v, p1_v, s0_v, s1_v, sem):
        wid = lax.axis_index("s") * NC + lax.axis_index("c")
        base = wid * TPW
        pltpu.sync_copy(pos0_hbm.at[pl.ds(base, TPW)], p0_v)
        pltpu.sync_copy(pos1_hbm.at[pl.ds(base, TPW)], p1_v)
        pltpu.sync_copy(sb0_hbm.at[pl.ds(base, TPW)], s0_v)
        pltpu.sync_copy(sb1_hbm.at[pl.ds(base, TPW)], s1_v)
        pltpu.async_copy(ys_hbm.at[p0_v], y0_v, sem).wait()
        pltpu.async_copy(ys_hbm.at[p1_v], y1_v, sem).wait()

        def body(i, carry):
            s0r = s0_v[i, :]
            s1r = s1_v[i, :]
            for j in range(D // 16):
                sl = pl.ds(j * 16, 16)
                y0_v[i, sl] = y0_v[i, sl] * s0r + y1_v[i, sl] * s1r
            return carry

        lax.fori_loop(0, TPW, body, 0)
        pltpu.sync_copy(y0_v, out_hbm.at[pl.ds(base, TPW)])

    return _dispatch, _combine


def kernel(x, gate_w, w1, w2, balance_bias):
    b, s, d = x.shape
    xf = x.reshape(-1, d)
    bias2d = balance_bias.reshape(1, E)
    pos0, pos1, sb0, sb1, eb, chg, slot, nxt, issue, nact = _route(
        xf, gate_w, bias2d)
    pos0 = pos0.reshape(-1)
    pos1 = pos1.reshape(-1)
    dispatch_fn, combine_fn = _sc_kernels()
    xs = dispatch_fn(xf, pos0, pos1)
    ys = _ffn(eb.reshape(-1), chg.reshape(-1), slot.reshape(-1),
              nxt.reshape(-1), issue.reshape(-1), nact.reshape(-1),
              xs, w1, w2)
    out = combine_fn(ys, pos0, pos1, sb0, sb1)
    return out.reshape(b, s, d)


# speedup vs baseline: 1.1016x; 1.0006x over previous
import functools

import jax
import jax.numpy as jnp
from jax import lax
from jax.experimental import pallas as pl
from jax.experimental.pallas import tpu as pltpu
from jax.experimental.pallas import tpu_sc as plsc

T = 2048
D = 768
E = 16
F = 1024
BLK = 256
NBLK = (2 * T) // BLK + E
CAP = NBLK * BLK
NC = 2
NS = 16
NW = NC * NS
TPW = T // NW


def _cumsum_rows(a, n):
    sh = 1
    while sh < n:
        z = jnp.zeros((sh, a.shape[1]), a.dtype)
        a = a + jnp.concatenate([z, a[:-sh, :]], axis=0)
        sh *= 2
    return a


def _route_body(x_ref, gw_ref, bias_ref, pos0_ref, pos1_ref, sb0_ref, sb1_ref,
                eb_ref, chg_ref, slot_ref, nxt_ref, issue_ref, nact_ref):
    x = x_ref[...]
    gw = gw_ref[...]
    logits = lax.dot_general(x, gw, (((1,), (1,)), ((), ())),
                             preferred_element_type=jnp.float32)
    scores = jax.nn.sigmoid(logits + bias_ref[...])

    eidx = lax.broadcasted_iota(jnp.int32, (T, E), 1)
    m0 = jnp.max(scores, axis=1, keepdims=True)
    i0 = jnp.min(jnp.where(scores == m0, eidx, E), axis=1, keepdims=True)
    masked = jnp.where(eidx == i0, -1.0, scores)
    m1 = jnp.max(masked, axis=1, keepdims=True)
    i1 = jnp.min(jnp.where(masked == m1, eidx, E), axis=1, keepdims=True)
    denom = m0 + m1 + 1e-6
    s0 = m0 / denom
    s1 = m1 / denom
    sb0_ref[...] = jnp.broadcast_to(s0, (T, E))
    sb1_ref[...] = jnp.broadcast_to(s1, (T, E))

    oh0 = (eidx == i0).astype(jnp.float32)
    oh1 = (eidx == i1).astype(jnp.float32)
    c01 = _cumsum_rows(jnp.concatenate([oh0, oh1], axis=1), T)
    c0 = c01[:, :E]
    c1 = c01[:, E:]

    counts = jnp.sum(oh0 + oh1, axis=0, keepdims=True)
    nb = jnp.floor((counts + (BLK - 1)) * (1.0 / BLK))
    r16 = lax.broadcasted_iota(jnp.int32, (E, E), 0)
    cjj = lax.broadcasted_iota(jnp.int32, (E, E), 1)
    lt_strict = (r16 < cjj).astype(jnp.float32)
    offs = lax.dot_general(nb, lt_strict, (((1,), (0,)), ((), ())),
                           preferred_element_type=jnp.float32)
    offs_end = offs + nb
    offset_pad = offs * float(BLK)

    rank0 = jnp.sum(oh0 * (c0 - 1.0 + c1), axis=1, keepdims=True)
    rank1 = jnp.sum(oh1 * (c0 + c1 - 1.0), axis=1, keepdims=True)
    base0 = jnp.sum(oh0 * offset_pad, axis=1, keepdims=True)
    base1 = jnp.sum(oh1 * offset_pad, axis=1, keepdims=True)
    pos0_ref[...] = (base0 + rank0).astype(jnp.int32)
    pos1_ref[...] = (base1 + rank1).astype(jnp.int32)

    gidx = lax.broadcasted_iota(jnp.int32, (NBLK, 1), 0).astype(jnp.float32)
    ebf = jnp.sum((jnp.broadcast_to(offs_end, (NBLK, E)) <= gidx)
                  .astype(jnp.float32), axis=1, keepdims=True)
    e_last = jnp.max(jnp.where(nb > 0.0,
                               lax.broadcasted_iota(jnp.int32, (1, E), 1)
                               .astype(jnp.float32),
                               -1.0), axis=1, keepdims=True)
    e_last = jnp.maximum(e_last, 0.0)
    ebf = jnp.minimum(ebf, jnp.broadcast_to(e_last, (NBLK, 1)))
    eb_ref[...] = ebf.astype(jnp.int32)
    nact_ref[...] = offs_end[:, E - 1:E].astype(jnp.int32)

    ebprev = jnp.concatenate([jnp.full((1, 1), -1.0, jnp.float32),
                              ebf[:-1, :]], axis=0)
    chg = (ebf != ebprev).astype(jnp.float32)
    runid = _cumsum_rows(chg, NBLK) - 1.0
    slot = runid - 2.0 * jnp.floor(runid * 0.5)
    r_iota = lax.broadcasted_iota(jnp.int32, (1, NBLK), 1).astype(jnp.float32)
    m_run = (runid == r_iota).astype(jnp.float32) * chg
    eor = jnp.sum(m_run * ebf, axis=0, keepdims=True)
    nruns = jnp.max(runid, axis=0, keepdims=True) + 1.0
    nxt = jnp.sum(((runid + 1.0) == r_iota).astype(jnp.float32) * eor,
                  axis=1, keepdims=True)
    issue = chg * ((runid + 1.0) < nruns).astype(jnp.float32)
    chg_ref[...] = chg.astype(jnp.int32)
    slot_ref[...] = slot.astype(jnp.int32)
    nxt_ref[...] = nxt.astype(jnp.int32)
    issue_ref[...] = issue.astype(jnp.int32)


def _route(xf, gate_w, bias2d, interpret=False):
    return pl.pallas_call(
        _route_body,
        out_shape=(
            jax.ShapeDtypeStruct((T, 1), jnp.int32),
            jax.ShapeDtypeStruct((T, 1), jnp.int32),
            jax.ShapeDtypeStruct((T, E), jnp.float32),
            jax.ShapeDtypeStruct((T, E), jnp.float32),
            jax.ShapeDtypeStruct((NBLK, 1), jnp.int32),
            jax.ShapeDtypeStruct((NBLK, 1), jnp.int32),
            jax.ShapeDtypeStruct((NBLK, 1), jnp.int32),
            jax.ShapeDtypeStruct((NBLK, 1), jnp.int32),
            jax.ShapeDtypeStruct((NBLK, 1), jnp.int32),
            jax.ShapeDtypeStruct((1, 1), jnp.int32),
        ),
        interpret=interpret,
    )(xf, gate_w, bias2d)


def _ffn_body(eb_s, chg_s, slot_s, nxt_s, issue_s, nact_s,
              xs_ref, w1_hbm, w2_hbm, y_ref, w1b, w2b, sem1, sem2):
    g = pl.program_id(0)
    s = slot_s[g]
    cur = eb_s[g]

    @pl.when(g == 0)
    def _():
        pltpu.make_async_copy(w1_hbm.at[cur], w1b.at[s], sem1.at[s]).start()
        pltpu.make_async_copy(w2_hbm.at[cur], w2b.at[s], sem2.at[s]).start()

    @pl.when(chg_s[g] == 1)
    def _():
        pltpu.make_async_copy(w1_hbm.at[cur], w1b.at[s], sem1.at[s]).wait()
        pltpu.make_async_copy(w2_hbm.at[cur], w2b.at[s], sem2.at[s]).wait()

    @pl.when(issue_s[g] == 1)
    def _():
        nx = nxt_s[g]
        pltpu.make_async_copy(w1_hbm.at[nx], w1b.at[1 - s], sem1.at[1 - s]).start()
        pltpu.make_async_copy(w2_hbm.at[nx], w2b.at[1 - s], sem2.at[1 - s]).start()

    @pl.when(g < nact_s[0])
    def _():
        xb = xs_ref[...].astype(jnp.bfloat16)
        h = lax.dot_general(xb, w1b[s].astype(jnp.bfloat16),
                            (((1,), (1,)), ((), ())),
                            preferred_element_type=jnp.float32)
        h = h * jax.nn.sigmoid(h)
        y_ref[...] = lax.dot_general(h.astype(jnp.bfloat16),
                                     w2b[s].astype(jnp.bfloat16),
                                     (((1,), (1,)), ((), ())),
                                     preferred_element_type=jnp.float32)


def _ffn(eb, chg, slot, nxt, issue, nact, xs, w1, w2, interpret=False):
    grid_spec = pltpu.PrefetchScalarGridSpec(
        num_scalar_prefetch=6,
        grid=(NBLK,),
        in_specs=[
            pl.BlockSpec((BLK, D), lambda g, *_: (g, 0)),
            pl.BlockSpec(memory_space=pltpu.MemorySpace.HBM),
            pl.BlockSpec(memory_space=pltpu.MemorySpace.HBM),
        ],
        out_specs=pl.BlockSpec((BLK, D), lambda g, *_: (g, 0)),
        scratch_shapes=[
            pltpu.VMEM((2, F, D), jnp.float32),
            pltpu.VMEM((2, D, F), jnp.float32),
            pltpu.SemaphoreType.DMA((2,)),
            pltpu.SemaphoreType.DMA((2,)),
        ],
    )
    return pl.pallas_call(
        _ffn_body,
        grid_spec=grid_spec,
        out_shape=jax.ShapeDtypeStruct((CAP, D), jnp.float32),
        interpret=interpret,
    )(eb, chg, slot, nxt, issue, nact, xs, w1, w2)


@functools.cache
def _sc_kernels():
    mesh = plsc.VectorSubcoreMesh(core_axis_name="c", subcore_axis_name="s",
                                  num_cores=NC, num_subcores=NS)

    @functools.partial(
        pl.kernel,
        out_type=jax.ShapeDtypeStruct((CAP, D), jnp.float32),
        mesh=mesh,
        scratch_types=[
            pltpu.VMEM((TPW, D), jnp.float32),
            pltpu.VMEM((TPW,), jnp.int32),
            pltpu.VMEM((TPW,), jnp.int32),
            pltpu.SemaphoreType.DMA,
        ],
    )
    def _dispatch(x_hbm, pos0_hbm, pos1_hbm, out_hbm, rows_v, p0_v, p1_v, sem):
        wid = lax.axis_index("s") * NC + lax.axis_index("c")
        base = wid * TPW
        pltpu.sync_copy(x_hbm.at[pl.ds(base, TPW)], rows_v)
        pltpu.sync_copy(pos0_hbm.at[pl.ds(base, TPW)], p0_v)
        pltpu.sync_copy(pos1_hbm.at[pl.ds(base, TPW)], p1_v)
        pltpu.async_copy(rows_v, out_hbm.at[p0_v], sem).wait()
        pltpu.async_copy(rows_v, out_hbm.at[p1_v], sem).wait()

    @functools.partial(
        pl.kernel,
        out_type=jax.ShapeDtypeStruct((T, D), jnp.float32),
        mesh=mesh,
        scratch_types=[
            pltpu.VMEM((TPW, D), jnp.float32),
            pltpu.VMEM((TPW, D), jnp.float32),
            pltpu.VMEM((TPW,), jnp.int32),
            pltpu.VMEM((TPW,), jnp.int32),
            pltpu.VMEM((TPW, E), jnp.float32),
            pltpu.VMEM((TPW, E), jnp.float32),
            pltpu.SemaphoreType.DMA,
        ],
    )
    def _combine(ys_hbm, pos0_hbm, pos1_hbm, sb0_hbm, sb1_hbm, out_hbm,
                 y0_v, y1_v, p0_v, p1_v, s0_v, s1_v, sem):
        wid = lax.axis_index("s") * NC + lax.axis_index("c")
        base = wid * TPW
        pltpu.sync_copy(pos0_hbm.at[pl.ds(base, TPW)], p0_v)
        pltpu.sync_copy(pos1_hbm.at[pl.ds(base, TPW)], p1_v)
        pltpu.sync_copy(sb0_hbm.at[pl.ds(base, TPW)], s0_v)
        pltpu.sync_copy(sb1_hbm.at[pl.ds(base, TPW)], s1_v)
        pltpu.async_copy(ys_hbm.at[p0_v], y0_v, sem).wait()
        pltpu.async_copy(ys_hbm.at[p1_v], y1_v, sem).wait()

        def body(i, carry):
            s0r = s0_v[i, :]
            s1r = s1_v[i, :]
            for j in range(D // 16):
                sl = pl.ds(j * 16, 16)
                y0_v[i, sl] = y0_v[i, sl] * s0r + y1_v[i, sl] * s1r
            return carry

        lax.fori_loop(0, TPW, body, 0)
        pltpu.sync_copy(y0_v, out_hbm.at[pl.ds(base, TPW)])

    return _dispatch, _combine


def kernel(x, gate_w, w1, w2, balance_bias):
    b, s, d = x.shape
    xf = x.reshape(-1, d)
    bias2d = balance_bias.reshape(1, E)
    pos0, pos1, sb0, sb1, eb, chg, slot, nxt, issue, nact = _route(
        xf, gate_w, bias2d)
    pos0 = pos0.reshape(-1)
    pos1 = pos1.reshape(-1)
    dispatch_fn, combine_fn = _sc_kernels()
    xs = dispatch_fn(xf, pos0, pos1)
    ys = _ffn(eb.reshape(-1), chg.reshape(-1), slot.reshape(-1),
              nxt.reshape(-1), issue.reshape(-1), nact.reshape(-1),
              xs, w1, w2)
    out = combine_fn(ys, pos0, pos1, sb0, sb1)
    return out.reshape(b, s, d)


# speedup vs baseline: 1.2467x; 1.1317x over previous
import functools

import jax
import jax.numpy as jnp
from jax import lax
from jax.experimental import pallas as pl
from jax.experimental.pallas import tpu as pltpu
from jax.experimental.pallas import tpu_sc as plsc

T = 2048
D = 768
E = 16
F = 1024
BLK = 256
WBUF = 3
NBLK = (2 * T) // BLK + E
CAP = NBLK * BLK
NC = 2
NS = 16
NW = NC * NS
TPW = T // NW


def _cumsum_rows(a, n):
    sh = 1
    while sh < n:
        z = jnp.zeros((sh, a.shape[1]), a.dtype)
        a = a + jnp.concatenate([z, a[:-sh, :]], axis=0)
        sh *= 2
    return a


def _route_body(x_ref, gw_ref, bias_ref, pos0_ref, pos1_ref, sb0_ref, sb1_ref,
                eb_ref, chg_ref, slot_ref, nxt_ref, issue_ref, nxt2_ref,
                issue2_ref, nact_ref):
    x = x_ref[...]
    gw = gw_ref[...]
    logits = lax.dot_general(x, gw, (((1,), (1,)), ((), ())),
                             preferred_element_type=jnp.float32)
    scores = jax.nn.sigmoid(logits + bias_ref[...])

    eidx = lax.broadcasted_iota(jnp.int32, (T, E), 1)
    m0 = jnp.max(scores, axis=1, keepdims=True)
    i0 = jnp.min(jnp.where(scores == m0, eidx, E), axis=1, keepdims=True)
    masked = jnp.where(eidx == i0, -1.0, scores)
    m1 = jnp.max(masked, axis=1, keepdims=True)
    i1 = jnp.min(jnp.where(masked == m1, eidx, E), axis=1, keepdims=True)
    denom = m0 + m1 + 1e-6
    s0 = m0 / denom
    s1 = m1 / denom
    sb0_ref[...] = jnp.broadcast_to(s0, (T, E))
    sb1_ref[...] = jnp.broadcast_to(s1, (T, E))

    oh0 = (eidx == i0).astype(jnp.float32)
    oh1 = (eidx == i1).astype(jnp.float32)
    c01 = _cumsum_rows(jnp.concatenate([oh0, oh1], axis=1), T)
    c0 = c01[:, :E]
    c1 = c01[:, E:]

    counts = jnp.sum(oh0 + oh1, axis=0, keepdims=True)
    nb = jnp.floor((counts + (BLK - 1)) * (1.0 / BLK))
    r16 = lax.broadcasted_iota(jnp.int32, (E, E), 0)
    cjj = lax.broadcasted_iota(jnp.int32, (E, E), 1)
    lt_strict = (r16 < cjj).astype(jnp.float32)
    offs = lax.dot_general(nb, lt_strict, (((1,), (0,)), ((), ())),
                           preferred_element_type=jnp.float32)
    offs_end = offs + nb
    offset_pad = offs * float(BLK)

    rank0 = jnp.sum(oh0 * (c0 - 1.0 + c1), axis=1, keepdims=True)
    rank1 = jnp.sum(oh1 * (c0 + c1 - 1.0), axis=1, keepdims=True)
    base0 = jnp.sum(oh0 * offset_pad, axis=1, keepdims=True)
    base1 = jnp.sum(oh1 * offset_pad, axis=1, keepdims=True)
    pos0_ref[...] = (base0 + rank0).astype(jnp.int32)
    pos1_ref[...] = (base1 + rank1).astype(jnp.int32)

    gidx = lax.broadcasted_iota(jnp.int32, (NBLK, 1), 0).astype(jnp.float32)
    ebf = jnp.sum((jnp.broadcast_to(offs_end, (NBLK, E)) <= gidx)
                  .astype(jnp.float32), axis=1, keepdims=True)
    e_last = jnp.max(jnp.where(nb > 0.0,
                               lax.broadcasted_iota(jnp.int32, (1, E), 1)
                               .astype(jnp.float32),
                               -1.0), axis=1, keepdims=True)
    e_last = jnp.maximum(e_last, 0.0)
    ebf = jnp.minimum(ebf, jnp.broadcast_to(e_last, (NBLK, 1)))
    eb_ref[...] = ebf.astype(jnp.int32)
    nact_ref[...] = offs_end[:, E - 1:E].astype(jnp.int32)

    ebprev = jnp.concatenate([jnp.full((1, 1), -1.0, jnp.float32),
                              ebf[:-1, :]], axis=0)
    chg = (ebf != ebprev).astype(jnp.float32)
    runid = _cumsum_rows(chg, NBLK) - 1.0
    slot = runid - float(WBUF) * jnp.floor(runid * (1.0 / WBUF))
    r_iota = lax.broadcasted_iota(jnp.int32, (1, NBLK), 1).astype(jnp.float32)
    m_run = (runid == r_iota).astype(jnp.float32) * chg
    eor = jnp.sum(m_run * ebf, axis=0, keepdims=True)
    nruns = jnp.max(runid, axis=0, keepdims=True) + 1.0
    nxt = jnp.sum(((runid + 1.0) == r_iota).astype(jnp.float32) * eor,
                  axis=1, keepdims=True)
    issue = chg * ((runid + 1.0) < nruns).astype(jnp.float32)
    la = float(WBUF - 1)
    nxt2 = jnp.sum(((runid + la) == r_iota).astype(jnp.float32) * eor,
                   axis=1, keepdims=True)
    issue2 = chg * ((runid + la) < nruns).astype(jnp.float32)
    chg_ref[...] = chg.astype(jnp.int32)
    slot_ref[...] = slot.astype(jnp.int32)
    nxt_ref[...] = nxt.astype(jnp.int32)
    issue_ref[...] = issue.astype(jnp.int32)
    nxt2_ref[...] = nxt2.astype(jnp.int32)
    issue2_ref[...] = issue2.astype(jnp.int32)


def _route(xf, gate_w, bias2d, interpret=False):
    return pl.pallas_call(
        _route_body,
        out_shape=(
            jax.ShapeDtypeStruct((T, 1), jnp.int32),
            jax.ShapeDtypeStruct((T, 1), jnp.int32),
            jax.ShapeDtypeStruct((T, E), jnp.float32),
            jax.ShapeDtypeStruct((T, E), jnp.float32),
            jax.ShapeDtypeStruct((NBLK, 1), jnp.int32),
            jax.ShapeDtypeStruct((NBLK, 1), jnp.int32),
            jax.ShapeDtypeStruct((NBLK, 1), jnp.int32),
            jax.ShapeDtypeStruct((NBLK, 1), jnp.int32),
            jax.ShapeDtypeStruct((NBLK, 1), jnp.int32),
            jax.ShapeDtypeStruct((NBLK, 1), jnp.int32),
            jax.ShapeDtypeStruct((NBLK, 1), jnp.int32),
            jax.ShapeDtypeStruct((1, 1), jnp.int32),
        ),
        interpret=interpret,
    )(xf, gate_w, bias2d)


def _ffn_body(eb_s, chg_s, slot_s, nxt_s, issue_s, nxt2_s, issue2_s, nact_s,
              xs_ref, w1_hbm, w2_hbm, y_ref, w1b, w2b, sem1, sem2):
    g = pl.program_id(0)
    s = slot_s[g]
    cur = eb_s[g]

    @pl.when(g == 0)
    def _():
        pltpu.make_async_copy(w1_hbm.at[cur], w1b.at[s], sem1.at[s]).start()
        pltpu.make_async_copy(w2_hbm.at[cur], w2b.at[s], sem2.at[s]).start()

    @pl.when(jnp.logical_and(g == 0, issue_s[0] == 1))
    def _():
        nx = nxt_s[0]
        pltpu.make_async_copy(w1_hbm.at[nx], w1b.at[1], sem1.at[1]).start()
        pltpu.make_async_copy(w2_hbm.at[nx], w2b.at[1], sem2.at[1]).start()

    @pl.when(chg_s[g] == 1)
    def _():
        pltpu.make_async_copy(w1_hbm.at[cur], w1b.at[s], sem1.at[s]).wait()
        pltpu.make_async_copy(w2_hbm.at[cur], w2b.at[s], sem2.at[s]).wait()

    @pl.when(issue2_s[g] == 1)
    def _():
        nx = nxt2_s[g]
        s2 = lax.rem(s + (WBUF - 1), WBUF)
        pltpu.make_async_copy(w1_hbm.at[nx], w1b.at[s2], sem1.at[s2]).start()
        pltpu.make_async_copy(w2_hbm.at[nx], w2b.at[s2], sem2.at[s2]).start()

    @pl.when(g < nact_s[0])
    def _():
        xb = xs_ref[...]
        h = lax.dot_general(xb, w1b[s], (((1,), (1,)), ((), ())),
                            preferred_element_type=jnp.float32)
        h = h * jax.nn.sigmoid(h)
        y_ref[...] = lax.dot_general(h, w2b[s], (((1,), (1,)), ((), ())),
                                     preferred_element_type=jnp.float32)


def _ffn(eb, chg, slot, nxt, issue, nxt2, issue2, nact, xs, w1, w2,
         interpret=False):
    grid_spec = pltpu.PrefetchScalarGridSpec(
        num_scalar_prefetch=8,
        grid=(NBLK,),
        in_specs=[
            pl.BlockSpec((BLK, D), lambda g, *_: (g, 0)),
            pl.BlockSpec(memory_space=pltpu.MemorySpace.HBM),
            pl.BlockSpec(memory_space=pltpu.MemorySpace.HBM),
        ],
        out_specs=pl.BlockSpec((BLK, D), lambda g, *_: (g, 0)),
        scratch_shapes=[
            pltpu.VMEM((WBUF, F, D), jnp.float32),
            pltpu.VMEM((WBUF, D, F), jnp.float32),
            pltpu.SemaphoreType.DMA((WBUF,)),
            pltpu.SemaphoreType.DMA((WBUF,)),
        ],
    )
    return pl.pallas_call(
        _ffn_body,
        grid_spec=grid_spec,
        out_shape=jax.ShapeDtypeStruct((CAP, D), jnp.float32),
        interpret=interpret,
    )(eb, chg, slot, nxt, issue, nxt2, issue2, nact, xs, w1, w2)


@functools.cache
def _sc_kernels():
    mesh = plsc.VectorSubcoreMesh(core_axis_name="c", subcore_axis_name="s",
                                  num_cores=NC, num_subcores=NS)

    @functools.partial(
        pl.kernel,
        out_type=jax.ShapeDtypeStruct((CAP, D), jnp.float32),
        mesh=mesh,
        scratch_types=[
            pltpu.VMEM((TPW, D), jnp.float32),
            pltpu.VMEM((TPW,), jnp.int32),
            pltpu.VMEM((TPW,), jnp.int32),
            pltpu.SemaphoreType.DMA,
        ],
    )
    def _dispatch(x_hbm, pos0_hbm, pos1_hbm, out_hbm, rows_v, p0_v, p1_v, sem):
        wid = lax.axis_index("s") * NC + lax.axis_index("c")
        base = wid * TPW
        pltpu.sync_copy(x_hbm.at[pl.ds(base, TPW)], rows_v)
        pltpu.sync_copy(pos0_hbm.at[pl.ds(base, TPW)], p0_v)
        pltpu.sync_copy(pos1_hbm.at[pl.ds(base, TPW)], p1_v)
        pltpu.async_copy(rows_v, out_hbm.at[p0_v], sem).wait()
        pltpu.async_copy(rows_v, out_hbm.at[p1_v], sem).wait()

    @functools.partial(
        pl.kernel,
        out_type=jax.ShapeDtypeStruct((T, D), jnp.float32),
        mesh=mesh,
        scratch_types=[
            pltpu.VMEM((TPW, D), jnp.float32),
            pltpu.VMEM((TPW, D), jnp.float32),
            pltpu.VMEM((TPW,), jnp.int32),
            pltpu.VMEM((TPW,), jnp.int32),
            pltpu.VMEM((TPW, E), jnp.float32),
            pltpu.VMEM((TPW, E), jnp.float32),
            pltpu.SemaphoreType.DMA,
        ],
    )
    def _combine(ys_hbm, pos0_hbm, pos1_hbm, sb0_hbm, sb1_hbm, out_hbm,
                 y0_v, y1_v, p0_v, p1_v, s0_v, s1_v, sem):
        wid = lax.axis_index("s") * NC + lax.axis_index("c")
        base = wid * TPW
        pltpu.sync_copy(pos0_hbm.at[pl.ds(base, TPW)], p0_v)
        pltpu.sync_copy(pos1_hbm.at[pl.ds(base, TPW)], p1_v)
        pltpu.sync_copy(sb0_hbm.at[pl.ds(base, TPW)], s0_v)
        pltpu.sync_copy(sb1_hbm.at[pl.ds(base, TPW)], s1_v)
        pltpu.async_copy(ys_hbm.at[p0_v], y0_v, sem).wait()
        pltpu.async_copy(ys_hbm.at[p1_v], y1_v, sem).wait()

        def body(i, carry):
            s0r = s0_v[i, :]
            s1r = s1_v[i, :]
            for j in range(D // 16):
                sl = pl.ds(j * 16, 16)
                y0_v[i, sl] = y0_v[i, sl] * s0r + y1_v[i, sl] * s1r
            return carry

        lax.fori_loop(0, TPW, body, 0)
        pltpu.sync_copy(y0_v, out_hbm.at[pl.ds(base, TPW)])

    return _dispatch, _combine


def kernel(x, gate_w, w1, w2, balance_bias):
    b, s, d = x.shape
    xf = x.reshape(-1, d)
    bias2d = balance_bias.reshape(1, E)
    (pos0, pos1, sb0, sb1, eb, chg, slot, nxt, issue, nxt2, issue2,
     nact) = _route(xf, gate_w, bias2d)
    pos0 = pos0.reshape(-1)
    pos1 = pos1.reshape(-1)
    dispatch_fn, combine_fn = _sc_kernels()
    xs = dispatch_fn(xf, pos0, pos1)
    ys = _ffn(eb.reshape(-1), chg.reshape(-1), slot.reshape(-1),
              nxt.reshape(-1), issue.reshape(-1), nxt2.reshape(-1),
              issue2.reshape(-1), nact.reshape(-1), xs, w1, w2)
    out = combine_fn(ys, pos0, pos1, sb0, sb1)
    return out.reshape(b, s, d)


# speedup vs baseline: 1.2468x; 1.0001x over previous
import functools

import jax
import jax.numpy as jnp
from jax import lax
from jax.experimental import pallas as pl
from jax.experimental.pallas import tpu as pltpu
from jax.experimental.pallas import tpu_sc as plsc

T = 2048
D = 768
E = 16
F = 1024
BLK = 256
WBUF = 3
NBLK = (2 * T) // BLK + E
CAP = NBLK * BLK
NC = 2
NS = 16
NW = NC * NS
TPW = T // NW


def _cumsum_rows(a, n):
    sh = 1
    while sh < n:
        z = jnp.zeros((sh, a.shape[1]), a.dtype)
        a = a + jnp.concatenate([z, a[:-sh, :]], axis=0)
        sh *= 2
    return a


def _route_body(x_ref, gw_ref, bias_ref, pos0_ref, pos1_ref, sb0_ref, sb1_ref,
                eb_ref, chg_ref, slot_ref, nxt_ref, issue_ref, nxt2_ref,
                issue2_ref, nact_ref):
    x = x_ref[...]
    gw = gw_ref[...]
    logits = lax.dot_general(x, gw, (((1,), (1,)), ((), ())),
                             preferred_element_type=jnp.float32)
    scores = jax.nn.sigmoid(logits + bias_ref[...])

    eidx = lax.broadcasted_iota(jnp.int32, (T, E), 1)
    m0 = jnp.max(scores, axis=1, keepdims=True)
    i0 = jnp.min(jnp.where(scores == m0, eidx, E), axis=1, keepdims=True)
    masked = jnp.where(eidx == i0, -1.0, scores)
    m1 = jnp.max(masked, axis=1, keepdims=True)
    i1 = jnp.min(jnp.where(masked == m1, eidx, E), axis=1, keepdims=True)
    denom = m0 + m1 + 1e-6
    s0 = m0 / denom
    s1 = m1 / denom
    sb0_ref[...] = jnp.broadcast_to(s0, (T, E))
    sb1_ref[...] = jnp.broadcast_to(s1, (T, E))

    oh0 = (eidx == i0).astype(jnp.float32)
    oh1 = (eidx == i1).astype(jnp.float32)
    c01 = _cumsum_rows(jnp.concatenate([oh0, oh1], axis=1), T)
    c0 = c01[:, :E]
    c1 = c01[:, E:]

    counts = jnp.sum(oh0 + oh1, axis=0, keepdims=True)
    nb = jnp.floor((counts + (BLK - 1)) * (1.0 / BLK))
    r16 = lax.broadcasted_iota(jnp.int32, (E, E), 0)
    cjj = lax.broadcasted_iota(jnp.int32, (E, E), 1)
    lt_strict = (r16 < cjj).astype(jnp.float32)
    offs = lax.dot_general(nb, lt_strict, (((1,), (0,)), ((), ())),
                           preferred_element_type=jnp.float32)
    offs_end = offs + nb
    offset_pad = offs * float(BLK)

    rank0 = jnp.sum(oh0 * (c0 - 1.0 + c1), axis=1, keepdims=True)
    rank1 = jnp.sum(oh1 * (c0 + c1 - 1.0), axis=1, keepdims=True)
    base0 = jnp.sum(oh0 * offset_pad, axis=1, keepdims=True)
    base1 = jnp.sum(oh1 * offset_pad, axis=1, keepdims=True)
    pos0_ref[...] = (base0 + rank0).astype(jnp.int32)
    pos1_ref[...] = (base1 + rank1).astype(jnp.int32)

    gidx = lax.broadcasted_iota(jnp.int32, (NBLK, 1), 0).astype(jnp.float32)
    ebf = jnp.sum((jnp.broadcast_to(offs_end, (NBLK, E)) <= gidx)
                  .astype(jnp.float32), axis=1, keepdims=True)
    e_last = jnp.max(jnp.where(nb > 0.0,
                               lax.broadcasted_iota(jnp.int32, (1, E), 1)
                               .astype(jnp.float32),
                               -1.0), axis=1, keepdims=True)
    e_last = jnp.maximum(e_last, 0.0)
    ebf = jnp.minimum(ebf, jnp.broadcast_to(e_last, (NBLK, 1)))
    eb_ref[...] = ebf.astype(jnp.int32)
    nact_ref[...] = offs_end[:, E - 1:E].astype(jnp.int32)

    ebprev = jnp.concatenate([jnp.full((1, 1), -1.0, jnp.float32),
                              ebf[:-1, :]], axis=0)
    chg = (ebf != ebprev).astype(jnp.float32)
    runid = _cumsum_rows(chg, NBLK) - 1.0
    slot = runid - float(WBUF) * jnp.floor(runid * (1.0 / WBUF))
    r_iota = lax.broadcasted_iota(jnp.int32, (1, NBLK), 1).astype(jnp.float32)
    m_run = (runid == r_iota).astype(jnp.float32) * chg
    eor = jnp.sum(m_run * ebf, axis=0, keepdims=True)
    nruns = jnp.max(runid, axis=0, keepdims=True) + 1.0
    nxt = jnp.sum(((runid + 1.0) == r_iota).astype(jnp.float32) * eor,
                  axis=1, keepdims=True)
    issue = chg * ((runid + 1.0) < nruns).astype(jnp.float32)
    la = float(WBUF - 1)
    nxt2 = jnp.sum(((runid + la) == r_iota).astype(jnp.float32) * eor,
                   axis=1, keepdims=True)
    issue2 = chg * ((runid + la) < nruns).astype(jnp.float32)
    chg_ref[...] = chg.astype(jnp.int32)
    slot_ref[...] = slot.astype(jnp.int32)
    nxt_ref[...] = nxt.astype(jnp.int32)
    issue_ref[...] = issue.astype(jnp.int32)
    nxt2_ref[...] = nxt2.astype(jnp.int32)
    issue2_ref[...] = issue2.astype(jnp.int32)


def _route(xf, gate_w, bias2d, interpret=False):
    return pl.pallas_call(
        _route_body,
        out_shape=(
            jax.ShapeDtypeStruct((T, 1), jnp.int32),
            jax.ShapeDtypeStruct((T, 1), jnp.int32),
            jax.ShapeDtypeStruct((T, E), jnp.float32),
            jax.ShapeDtypeStruct((T, E), jnp.float32),
            jax.ShapeDtypeStruct((NBLK, 1), jnp.int32),
            jax.ShapeDtypeStruct((NBLK, 1), jnp.int32),
            jax.ShapeDtypeStruct((NBLK, 1), jnp.int32),
            jax.ShapeDtypeStruct((NBLK, 1), jnp.int32),
            jax.ShapeDtypeStruct((NBLK, 1), jnp.int32),
            jax.ShapeDtypeStruct((NBLK, 1), jnp.int32),
            jax.ShapeDtypeStruct((NBLK, 1), jnp.int32),
            jax.ShapeDtypeStruct((1, 1), jnp.int32),
        ),
        interpret=interpret,
    )(xf, gate_w, bias2d)


def _ffn_body(eb_s, chg_s, slot_s, nxt_s, issue_s, nxt2_s, issue2_s, nact_s,
              xs_ref, w1_hbm, w2_hbm, y_ref, w1b, w2b, sem1, sem2):
    g = pl.program_id(0)
    s = slot_s[g]
    cur = eb_s[g]

    @pl.when(g == 0)
    def _():
        pltpu.make_async_copy(w1_hbm.at[cur], w1b.at[s], sem1.at[s]).start()
        pltpu.make_async_copy(w2_hbm.at[cur], w2b.at[s], sem2.at[s]).start()

    @pl.when(jnp.logical_and(g == 0, issue_s[0] == 1))
    def _():
        nx = nxt_s[0]
        pltpu.make_async_copy(w1_hbm.at[nx], w1b.at[1], sem1.at[1]).start()
        pltpu.make_async_copy(w2_hbm.at[nx], w2b.at[1], sem2.at[1]).start()

    @pl.when(chg_s[g] == 1)
    def _():
        pltpu.make_async_copy(w1_hbm.at[cur], w1b.at[s], sem1.at[s]).wait()
        pltpu.make_async_copy(w2_hbm.at[cur], w2b.at[s], sem2.at[s]).wait()

    @pl.when(issue2_s[g] == 1)
    def _():
        nx = nxt2_s[g]
        s2 = lax.rem(s + (WBUF - 1), WBUF)
        pltpu.make_async_copy(w1_hbm.at[nx], w1b.at[s2], sem1.at[s2]).start()
        pltpu.make_async_copy(w2_hbm.at[nx], w2b.at[s2], sem2.at[s2]).start()

    @pl.when(g < nact_s[0])
    def _():
        xb = xs_ref[...]
        h = lax.dot_general(xb, w1b[s], (((1,), (1,)), ((), ())),
                            preferred_element_type=jnp.float32)
        h = h * jax.nn.sigmoid(h)
        y_ref[...] = lax.dot_general(h, w2b[s], (((1,), (1,)), ((), ())),
                                     preferred_element_type=jnp.float32)


def _ffn(eb, chg, slot, nxt, issue, nxt2, issue2, nact, xs, w1, w2,
         interpret=False):
    grid_spec = pltpu.PrefetchScalarGridSpec(
        num_scalar_prefetch=8,
        grid=(NBLK,),
        in_specs=[
            pl.BlockSpec((BLK, D),
                         lambda g, *pref: (jnp.minimum(g, pref[-1][0] - 1), 0)),
            pl.BlockSpec(memory_space=pltpu.MemorySpace.HBM),
            pl.BlockSpec(memory_space=pltpu.MemorySpace.HBM),
        ],
        out_specs=pl.BlockSpec(
            (BLK, D), lambda g, *pref: (jnp.minimum(g, pref[-1][0] - 1), 0)),
        scratch_shapes=[
            pltpu.VMEM((WBUF, F, D), jnp.float32),
            pltpu.VMEM((WBUF, D, F), jnp.float32),
            pltpu.SemaphoreType.DMA((WBUF,)),
            pltpu.SemaphoreType.DMA((WBUF,)),
        ],
    )
    return pl.pallas_call(
        _ffn_body,
        grid_spec=grid_spec,
        out_shape=jax.ShapeDtypeStruct((CAP, D), jnp.float32),
        interpret=interpret,
    )(eb, chg, slot, nxt, issue, nxt2, issue2, nact, xs, w1, w2)


@functools.cache
def _sc_kernels():
    mesh = plsc.VectorSubcoreMesh(core_axis_name="c", subcore_axis_name="s",
                                  num_cores=NC, num_subcores=NS)

    @functools.partial(
        pl.kernel,
        out_type=jax.ShapeDtypeStruct((CAP, D), jnp.float32),
        mesh=mesh,
        scratch_types=[
            pltpu.VMEM((TPW, D), jnp.float32),
            pltpu.VMEM((TPW,), jnp.int32),
            pltpu.VMEM((TPW,), jnp.int32),
            pltpu.SemaphoreType.DMA,
        ],
    )
    def _dispatch(x_hbm, pos0_hbm, pos1_hbm, out_hbm, rows_v, p0_v, p1_v, sem):
        wid = lax.axis_index("s") * NC + lax.axis_index("c")
        base = wid * TPW
        pltpu.sync_copy(x_hbm.at[pl.ds(base, TPW)], rows_v)
        pltpu.sync_copy(pos0_hbm.at[pl.ds(base, TPW)], p0_v)
        pltpu.sync_copy(pos1_hbm.at[pl.ds(base, TPW)], p1_v)
        pltpu.async_copy(rows_v, out_hbm.at[p0_v], sem).wait()
        pltpu.async_copy(rows_v, out_hbm.at[p1_v], sem).wait()

    @functools.partial(
        pl.kernel,
        out_type=jax.ShapeDtypeStruct((T, D), jnp.float32),
        mesh=mesh,
        scratch_types=[
            pltpu.VMEM((TPW, D), jnp.float32),
            pltpu.VMEM((TPW, D), jnp.float32),
            pltpu.VMEM((TPW,), jnp.int32),
            pltpu.VMEM((TPW,), jnp.int32),
            pltpu.VMEM((TPW, E), jnp.float32),
            pltpu.VMEM((TPW, E), jnp.float32),
            pltpu.SemaphoreType.DMA,
        ],
    )
    def _combine(ys_hbm, pos0_hbm, pos1_hbm, sb0_hbm, sb1_hbm, out_hbm,
                 y0_v, y1_v, p0_v, p1_v, s0_v, s1_v, sem):
        wid = lax.axis_index("s") * NC + lax.axis_index("c")
        base = wid * TPW
        pltpu.sync_copy(pos0_hbm.at[pl.ds(base, TPW)], p0_v)
        pltpu.sync_copy(pos1_hbm.at[pl.ds(base, TPW)], p1_v)
        pltpu.sync_copy(sb0_hbm.at[pl.ds(base, TPW)], s0_v)
        pltpu.sync_copy(sb1_hbm.at[pl.ds(base, TPW)], s1_v)
        pltpu.async_copy(ys_hbm.at[p0_v], y0_v, sem).wait()
        pltpu.async_copy(ys_hbm.at[p1_v], y1_v, sem).wait()

        def body(i, carry):
            s0r = s0_v[i, :]
            s1r = s1_v[i, :]
            for j in range(D // 16):
                sl = pl.ds(j * 16, 16)
                y0_v[i, sl] = y0_v[i, sl] * s0r + y1_v[i, sl] * s1r
            return carry

        lax.fori_loop(0, TPW, body, 0)
        pltpu.sync_copy(y0_v, out_hbm.at[pl.ds(base, TPW)])

    return _dispatch, _combine


def kernel(x, gate_w, w1, w2, balance_bias):
    b, s, d = x.shape
    xf = x.reshape(-1, d)
    bias2d = balance_bias.reshape(1, E)
    (pos0, pos1, sb0, sb1, eb, chg, slot, nxt, issue, nxt2, issue2,
     nact) = _route(xf, gate_w, bias2d)
    pos0 = pos0.reshape(-1)
    pos1 = pos1.reshape(-1)
    dispatch_fn, combine_fn = _sc_kernels()
    xs = dispatch_fn(xf, pos0, pos1)
    ys = _ffn(eb.reshape(-1), chg.reshape(-1), slot.reshape(-1),
              nxt.reshape(-1), issue.reshape(-1), nxt2.reshape(-1),
              issue2.reshape(-1), nact.reshape(-1), xs, w1, w2)
    out = combine_fn(ys, pos0, pos1, sb0, sb1)
    return out.reshape(b, s, d)
